# Initial kernel scaffold; baseline (speedup 1.0000x reference)
import jax
import jax.numpy as jnp
from jax.experimental import pallas as pl


def kernel(x, edge_index, Wc, bc, Wf0, bf0, Wf1, bf1, Wf2, bf2):
    raise NotImplementedError("write your pallas kernel here")


# speedup vs baseline: 10.0905x; 10.0905x over previous
import functools

import jax
import jax.numpy as jnp
from jax import lax
from jax.experimental import pallas as pl
from jax.experimental.pallas import tpu as pltpu
from jax.experimental.pallas import tpu_sc as plsc

N = 10000
D = 128
E = 320000
NCONV = 8

NC = 2
NS = 16
NW = NC * NS
EPW = E // NW
CH = 80
NCH = EPW // CH
NPT = 624
TOFF = NS * NPT
TAIL = N - TOFF

_f32 = jnp.float32


def _mesh():
    return plsc.VectorSubcoreMesh(core_axis_name="c", subcore_axis_name="s")


def _sc_degree(dst3):
    @functools.partial(
        pl.kernel,
        out_type=jax.ShapeDtypeStruct((NC, N), _f32),
        mesh=_mesh(),
        scratch_types=[
            pltpu.VMEM((NCH, CH), jnp.int32),
            pltpu.VMEM((CH,), _f32),
            pltpu.VMEM((N,), _f32),
            pltpu.VMEM_SHARED((N,), _f32),
        ],
    )
    def body(dst_hbm, out_hbm, dst_v, ones_v, zbuf, acc):
        c = lax.axis_index("c")
        s = lax.axis_index("s")
        wid = s * NC + c
        for j in range(CH // 16):
            ones_v[pl.ds(j * 16, 16)] = jnp.ones((16,), _f32)

        @pl.when(s == 0)
        def _zero():
            z16 = jnp.zeros((16,), _f32)

            def zstep(i, _):
                zbuf[pl.ds(i * 16, 16)] = z16
                return ()

            lax.fori_loop(0, N // 16, zstep, ())
            pltpu.sync_copy(zbuf, acc)

        pltpu.sync_copy(dst_hbm.at[wid], dst_v)
        plsc.subcore_barrier()

        def step(i, _):
            pltpu.sync_copy(ones_v, acc.at[dst_v.at[i]], add=True)
            return ()

        lax.fori_loop(0, NCH, step, ())
        plsc.subcore_barrier()

        @pl.when(s == 0)
        def _out():
            pltpu.sync_copy(acc, out_hbm.at[c])

    return body(dst3)


def _sc_segsum(u, src3, dst3):
    @functools.partial(
        pl.kernel,
        out_type=jax.ShapeDtypeStruct((NC, N, D), _f32),
        mesh=_mesh(),
        scratch_types=[
            pltpu.VMEM((NCH, CH), jnp.int32),
            pltpu.VMEM((NCH, CH), jnp.int32),
            pltpu.VMEM((CH, D), _f32),
            pltpu.VMEM_SHARED((N, D), _f32),
            pltpu.SemaphoreType.DMA,
        ],
    )
    def body(u_hbm, src_hbm, dst_hbm, out_hbm, src_v, dst_v, rows_v, acc, sem):
        c = lax.axis_index("c")
        s = lax.axis_index("s")
        wid = s * NC + c
        pltpu.sync_copy(u_hbm.at[pl.ds(s * NPT, NPT)], acc.at[pl.ds(s * NPT, NPT)])

        @pl.when(s == NS - 1)
        def _init_tail():
            pltpu.sync_copy(u_hbm.at[pl.ds(TOFF, TAIL)], acc.at[pl.ds(TOFF, TAIL)])

        pltpu.sync_copy(src_hbm.at[wid], src_v)
        pltpu.sync_copy(dst_hbm.at[wid], dst_v)
        plsc.subcore_barrier()

        def step(i, _):
            pltpu.async_copy(u_hbm.at[src_v.at[i]], rows_v, sem).wait()
            pltpu.sync_copy(rows_v, acc.at[dst_v.at[i]], add=True)
            return ()

        lax.fori_loop(0, NCH, step, ())
        plsc.subcore_barrier()
        pltpu.sync_copy(acc.at[pl.ds(s * NPT, NPT)], out_hbm.at[c, pl.ds(s * NPT, NPT)])

        @pl.when(s == NS - 1)
        def _out_tail():
            pltpu.sync_copy(acc.at[pl.ds(TOFF, TAIL)], out_hbm.at[c, pl.ds(TOFF, TAIL)])

    return body(u, src3, dst3)


def _sc_edge_gather(A, B, src3, dst3):
    @functools.partial(
        pl.kernel,
        out_type=(
            jax.ShapeDtypeStruct((E, D), _f32),
            jax.ShapeDtypeStruct((E, D), _f32),
        ),
        mesh=_mesh(),
        scratch_types=[
            pltpu.VMEM((NCH, CH), jnp.int32),
            pltpu.VMEM((NCH, CH), jnp.int32),
            pltpu.VMEM((CH, D), _f32),
            pltpu.VMEM((CH, D), _f32),
            pltpu.SemaphoreType.DMA,
            pltpu.SemaphoreType.DMA,
        ],
    )
    def body(A_hbm, B_hbm, row_hbm, col_hbm, gA_hbm, gB_hbm,
             row_v, col_v, bufA, bufB, semA, semB):
        c = lax.axis_index("c")
        s = lax.axis_index("s")
        wid = s * NC + c
        base = wid * EPW
        pltpu.sync_copy(row_hbm.at[wid], row_v)
        pltpu.sync_copy(col_hbm.at[wid], col_v)

        def step(i, _):
            da = pltpu.async_copy(A_hbm.at[row_v.at[i]], bufA, semA)
            db = pltpu.async_copy(B_hbm.at[col_v.at[i]], bufB, semB)
            da.wait()
            db.wait()
            eb = base + i * CH
            pltpu.sync_copy(bufA, gA_hbm.at[pl.ds(eb, CH)])
            pltpu.sync_copy(bufB, gB_hbm.at[pl.ds(eb, CH)])
            return ()

        lax.fori_loop(0, NCH, step, ())

    return body(A, B, src3, dst3)


RB = 2000
RE = 2000


def _tc_first(degP, x, W0):
    def body(degP_ref, x_ref, w_ref, dinv_ref, u_ref):
        deg = 1.0 + degP_ref[0] + degP_ref[1]
        dinv = lax.rsqrt(deg)
        dinv_ref[...] = dinv
        u_ref[...] = (
            jnp.dot(x_ref[...], w_ref[...], preferred_element_type=_f32) * dinv
        )

    return pl.pallas_call(
        body,
        grid=(N // RB,),
        in_specs=[
            pl.BlockSpec((NC, RB, 1), lambda i: (0, i, 0)),
            pl.BlockSpec((RB, D), lambda i: (i, 0)),
            pl.BlockSpec((D, D), lambda i: (0, 0)),
        ],
        out_specs=[
            pl.BlockSpec((RB, 1), lambda i: (i, 0)),
            pl.BlockSpec((RB, D), lambda i: (i, 0)),
        ],
        out_shape=[
            jax.ShapeDtypeStruct((N, 1), _f32),
            jax.ShapeDtypeStruct((N, D), _f32),
        ],
    )(degP, x, W0)


def _tc_mid(P, u, dinv, b, hprev, Wnext, first):
    def body(*refs):
        if first:
            P_ref, u_ref, dinv_ref, b_ref, w_ref, h_ref, un_ref = refs
        else:
            P_ref, u_ref, dinv_ref, b_ref, hp_ref, w_ref, h_ref, un_ref = refs
        t = (P_ref[0] + P_ref[1] - u_ref[...]) * dinv_ref[...] + b_ref[...]
        if not first:
            t = t + hp_ref[...]
        h = jnp.maximum(t, 0.0)
        h_ref[...] = h
        un_ref[...] = (
            jnp.dot(h, w_ref[...], preferred_element_type=_f32) * dinv_ref[...]
        )

    in_specs = [
        pl.BlockSpec((NC, RB, D), lambda i: (0, i, 0)),
        pl.BlockSpec((RB, D), lambda i: (i, 0)),
        pl.BlockSpec((RB, 1), lambda i: (i, 0)),
        pl.BlockSpec((1, D), lambda i: (0, 0)),
    ]
    args = [P, u, dinv, b]
    if not first:
        in_specs.append(pl.BlockSpec((RB, D), lambda i: (i, 0)))
        args.append(hprev)
    in_specs.append(pl.BlockSpec((D, D), lambda i: (0, 0)))
    args.append(Wnext)
    return pl.pallas_call(
        body,
        grid=(N // RB,),
        in_specs=in_specs,
        out_specs=[
            pl.BlockSpec((RB, D), lambda i: (i, 0)),
            pl.BlockSpec((RB, D), lambda i: (i, 0)),
        ],
        out_shape=[
            jax.ShapeDtypeStruct((N, D), _f32),
            jax.ShapeDtypeStruct((N, D), _f32),
        ],
    )(*args)


def _tc_last(P, u, dinv, b, hprev, Wf0pair):
    def body(P_ref, u_ref, dinv_ref, b_ref, hp_ref, w_ref, A_ref, B_ref):
        t = (P_ref[0] + P_ref[1] - u_ref[...]) * dinv_ref[...] + b_ref[...]
        h = jnp.maximum(t + hp_ref[...], 0.0)
        A_ref[...] = jnp.dot(h, w_ref[0], preferred_element_type=_f32)
        B_ref[...] = jnp.dot(h, w_ref[1], preferred_element_type=_f32)

    return pl.pallas_call(
        body,
        grid=(N // RB,),
        in_specs=[
            pl.BlockSpec((NC, RB, D), lambda i: (0, i, 0)),
            pl.BlockSpec((RB, D), lambda i: (i, 0)),
            pl.BlockSpec((RB, 1), lambda i: (i, 0)),
            pl.BlockSpec((1, D), lambda i: (0, 0)),
            pl.BlockSpec((RB, D), lambda i: (i, 0)),
            pl.BlockSpec((2, D, D), lambda i: (0, 0, 0)),
        ],
        out_specs=[
            pl.BlockSpec((RB, D), lambda i: (i, 0)),
            pl.BlockSpec((RB, D), lambda i: (i, 0)),
        ],
        out_shape=[
            jax.ShapeDtypeStruct((N, D), _f32),
            jax.ShapeDtypeStruct((N, D), _f32),
        ],
    )(P, u, dinv, b, hprev, Wf0pair)


def _tc_edge(gA, gB, b0, W1, b1, W2, b2):
    def body(gA_ref, gB_ref, b0_ref, w1_ref, b1_ref, w2_ref, b2_ref, out_ref):
        e = jnp.maximum(gA_ref[...] + gB_ref[...] + b0_ref[...], 0.0)
        e2 = jnp.maximum(
            jnp.dot(e, w1_ref[...], preferred_element_type=_f32) + b1_ref[...], 0.0
        )
        out_ref[...] = (
            jnp.dot(e2, w2_ref[...], preferred_element_type=_f32) + b2_ref[...]
        )

    return pl.pallas_call(
        body,
        grid=(E // RE,),
        in_specs=[
            pl.BlockSpec((RE, D), lambda i: (i, 0)),
            pl.BlockSpec((RE, D), lambda i: (i, 0)),
            pl.BlockSpec((1, D), lambda i: (0, 0)),
            pl.BlockSpec((D, D), lambda i: (0, 0)),
            pl.BlockSpec((1, D), lambda i: (0, 0)),
            pl.BlockSpec((D, 1), lambda i: (0, 0)),
            pl.BlockSpec((1, 1), lambda i: (0, 0)),
        ],
        out_specs=pl.BlockSpec((RE, 1), lambda i: (i, 0)),
        out_shape=jax.ShapeDtypeStruct((E, 1), _f32),
    )(gA, gB, b0, W1, b1, W2, b2)


def kernel(x, edge_index, Wc, bc, Wf0, bf0, Wf1, bf1, Wf2, bf2):
    src = edge_index[0].astype(jnp.int32)
    dst = edge_index[1].astype(jnp.int32)
    src3 = src.reshape(NW, NCH, CH)
    dst3 = dst.reshape(NW, NCH, CH)

    degP = _sc_degree(dst3).reshape(NC, N, 1)
    dinv, u = _tc_first(degP, x, Wc[0])

    h = None
    for i in range(NCONV):
        P = _sc_segsum(u, src3, dst3)
        bi = bc[i].reshape(1, D)
        if i < NCONV - 1:
            h, u = _tc_mid(P, u, dinv, bi, h, Wc[i + 1], first=(i == 0))
        else:
            A, B = _tc_last(P, u, dinv, bi, h, Wf0.reshape(2, D, D))

    gA, gB = _sc_edge_gather(A, B, src3, dst3)
    return _tc_edge(
        gA, gB, bf0.reshape(1, D), Wf1, bf1.reshape(1, D), Wf2, bf2.reshape(1, 1)
    )


# speedup vs baseline: 12.7006x; 1.2587x over previous
import functools

import jax
import jax.numpy as jnp
from jax import lax
from jax.experimental import pallas as pl
from jax.experimental.pallas import tpu as pltpu
from jax.experimental.pallas import tpu_sc as plsc

N = 10000
D = 128
E = 320000
NCONV = 8

NC = 2
NS = 16
NW = NC * NS
EPW = E // NW
CH = 80
NCH = EPW // CH
NPT = 624
TOFF = NS * NPT
TAIL = N - TOFF

_f32 = jnp.float32


def _mesh():
    return plsc.VectorSubcoreMesh(core_axis_name="c", subcore_axis_name="s")


def _sc_degree(dst3):
    @functools.partial(
        pl.kernel,
        out_type=jax.ShapeDtypeStruct((NW, N), _f32),
        mesh=_mesh(),
        compiler_params=pltpu.CompilerParams(needs_layout_passes=False),
        scratch_types=[
            pltpu.VMEM((EPW,), jnp.int32),
            pltpu.VMEM((N,), _f32),
        ],
    )
    def body(dst_hbm, out_hbm, dst_v, acc_t):
        c = lax.axis_index("c")
        s = lax.axis_index("s")
        wid = s * NC + c
        pltpu.sync_copy(dst_hbm.at[wid], dst_v)
        z16 = jnp.zeros((16,), _f32)

        def zstep(i, _):
            acc_t[pl.ds(i * 16, 16)] = z16
            return ()

        lax.fori_loop(0, N // 16, zstep, ())
        ones16 = jnp.ones((16,), _f32)

        def step(i, _):
            for j in range(4):
                idx = dst_v[pl.ds(i * 64 + j * 16, 16)]
                plsc.addupdate_scatter(acc_t, [idx], ones16)
            return ()

        lax.fori_loop(0, EPW // 64, step, ())
        pltpu.sync_copy(acc_t, out_hbm.at[wid])

    return body(dst3)


def _sc_segsum(u, src2, dst2):
    @functools.partial(
        pl.kernel,
        out_type=jax.ShapeDtypeStruct((NC, N, D), _f32),
        mesh=_mesh(),
        scratch_types=[
            pltpu.VMEM((EPW,), jnp.int32),
            pltpu.VMEM((EPW,), jnp.int32),
            pltpu.VMEM((2, CH), jnp.int32),
            pltpu.VMEM((2, CH, D), _f32),
            pltpu.VMEM_SHARED((N, D), _f32),
            pltpu.SemaphoreType.DMA,
            pltpu.SemaphoreType.DMA,
        ],
    )
    def body(u_hbm, src_hbm, dst_hbm, out_hbm, src_v, dst_v, dstage, rows_v,
             acc, s0, s1):
        sems = (s0, s1)
        c = lax.axis_index("c")
        s = lax.axis_index("s")
        wid = s * NC + c
        pltpu.sync_copy(u_hbm.at[pl.ds(s * NPT, NPT)], acc.at[pl.ds(s * NPT, NPT)])

        @pl.when(s == NS - 1)
        def _init_tail():
            pltpu.sync_copy(u_hbm.at[pl.ds(TOFF, TAIL)], acc.at[pl.ds(TOFF, TAIL)])

        pltpu.sync_copy(src_hbm.at[wid], src_v)
        pltpu.sync_copy(dst_hbm.at[wid], dst_v)
        plsc.subcore_barrier()

        def stage_dst(p, cc):
            for t in range(CH // 16):
                dstage[p, pl.ds(t * 16, 16)] = dst_v[pl.ds(cc * CH + t * 16, 16)]

        stage_dst(0, 0)
        pltpu.async_copy(
            u_hbm.at[src_v.at[pl.ds(0, CH)]], rows_v.at[0], sems[0]
        ).wait()

        def group(k, _):
            for j in (0, 1):
                cc = 2 * k + j
                jn = 1 - j
                d = pltpu.async_copy(
                    u_hbm.at[src_v.at[pl.ds((cc + 1) * CH, CH)]],
                    rows_v.at[jn], sems[jn],
                )
                stage_dst(jn, cc + 1)
                pltpu.sync_copy(rows_v.at[j], acc.at[dstage.at[j]], add=True)
                d.wait()
            return ()

        lax.fori_loop(0, (NCH - 1) // 2, group, ())
        pltpu.sync_copy(rows_v.at[0], acc.at[dstage.at[0]], add=True)
        plsc.subcore_barrier()
        pltpu.sync_copy(acc.at[pl.ds(s * NPT, NPT)], out_hbm.at[c, pl.ds(s * NPT, NPT)])

        @pl.when(s == NS - 1)
        def _out_tail():
            pltpu.sync_copy(acc.at[pl.ds(TOFF, TAIL)], out_hbm.at[c, pl.ds(TOFF, TAIL)])

    return body(u, src2, dst2)


def _sc_edge_gather(A, B, src3, dst3):
    @functools.partial(
        pl.kernel,
        out_type=(
            jax.ShapeDtypeStruct((E, D), _f32),
            jax.ShapeDtypeStruct((E, D), _f32),
        ),
        mesh=_mesh(),
        scratch_types=[
            pltpu.VMEM((NCH, CH), jnp.int32),
            pltpu.VMEM((NCH, CH), jnp.int32),
            pltpu.VMEM((2, CH, D), _f32),
            pltpu.VMEM((2, CH, D), _f32),
            pltpu.SemaphoreType.DMA,
            pltpu.SemaphoreType.DMA,
            pltpu.SemaphoreType.DMA,
            pltpu.SemaphoreType.DMA,
        ],
    )
    def body(A_hbm, B_hbm, row_hbm, col_hbm, gA_hbm, gB_hbm,
             row_v, col_v, bufA, bufB, sA0, sA1, sB0, sB1):
        sA = (sA0, sA1)
        sB = (sB0, sB1)
        c = lax.axis_index("c")
        s = lax.axis_index("s")
        wid = s * NC + c
        base = wid * EPW
        pltpu.sync_copy(row_hbm.at[wid], row_v)
        pltpu.sync_copy(col_hbm.at[wid], col_v)
        pltpu.async_copy(A_hbm.at[row_v.at[0]], bufA.at[0], sA[0])
        pltpu.async_copy(B_hbm.at[col_v.at[0]], bufB.at[0], sB[0])

        def group(k, _):
            for p in (1, 0):
                cc = 2 * k + (1 if p == 1 else 2)
                q = 1 - p
                pltpu.async_copy(A_hbm.at[row_v.at[cc]], bufA.at[p], sA[p])
                pltpu.async_copy(B_hbm.at[col_v.at[cc]], bufB.at[p], sB[p])
                pltpu.make_async_copy(
                    A_hbm.at[row_v.at[0]], bufA.at[q], sA[q]).wait()
                pltpu.make_async_copy(
                    B_hbm.at[col_v.at[0]], bufB.at[q], sB[q]).wait()
                eb = base + (cc - 1) * CH
                pltpu.sync_copy(bufA.at[q], gA_hbm.at[pl.ds(eb, CH)])
                pltpu.sync_copy(bufB.at[q], gB_hbm.at[pl.ds(eb, CH)])
            return ()

        lax.fori_loop(0, (NCH - 1) // 2, group, ())
        pltpu.make_async_copy(A_hbm.at[row_v.at[0]], bufA.at[0], sA[0]).wait()
        pltpu.make_async_copy(B_hbm.at[col_v.at[0]], bufB.at[0], sB[0]).wait()
        eb = base + (NCH - 1) * CH
        pltpu.sync_copy(bufA.at[0], gA_hbm.at[pl.ds(eb, CH)])
        pltpu.sync_copy(bufB.at[0], gB_hbm.at[pl.ds(eb, CH)])

    return body(A, B, src3, dst3)


RB = 2000
RE = 2000


def _tc_deg(degP):
    def body(degP_ref, dinv_ref):
        deg = 1.0 + jnp.sum(degP_ref[...], axis=0)
        dinv_ref[...] = lax.rsqrt(deg)[:, None]

    return pl.pallas_call(
        body,
        grid=(1,),
        in_specs=[pl.BlockSpec((NW, N), lambda i: (0, 0))],
        out_specs=pl.BlockSpec((N, 1), lambda i: (0, 0)),
        out_shape=jax.ShapeDtypeStruct((N, 1), _f32),
    )(degP)


def _tc_first(dinv, x, W0):
    def body(dinv_ref, x_ref, w_ref, u_ref):
        u_ref[...] = (
            jnp.dot(x_ref[...], w_ref[...], preferred_element_type=_f32)
            * dinv_ref[...]
        )

    return pl.pallas_call(
        body,
        grid=(N // RB,),
        in_specs=[
            pl.BlockSpec((RB, 1), lambda i: (i, 0)),
            pl.BlockSpec((RB, D), lambda i: (i, 0)),
            pl.BlockSpec((D, D), lambda i: (0, 0)),
        ],
        out_specs=pl.BlockSpec((RB, D), lambda i: (i, 0)),
        out_shape=jax.ShapeDtypeStruct((N, D), _f32),
    )(dinv, x, W0)


def _tc_mid(P, u, dinv, b, hprev, Wnext, first):
    def body(*refs):
        if first:
            P_ref, u_ref, dinv_ref, b_ref, w_ref, h_ref, un_ref = refs
        else:
            P_ref, u_ref, dinv_ref, b_ref, hp_ref, w_ref, h_ref, un_ref = refs
        t = (P_ref[0] + P_ref[1] - u_ref[...]) * dinv_ref[...] + b_ref[...]
        if not first:
            t = t + hp_ref[...]
        h = jnp.maximum(t, 0.0)
        h_ref[...] = h
        un_ref[...] = (
            jnp.dot(h, w_ref[...], preferred_element_type=_f32) * dinv_ref[...]
        )

    in_specs = [
        pl.BlockSpec((NC, RB, D), lambda i: (0, i, 0)),
        pl.BlockSpec((RB, D), lambda i: (i, 0)),
        pl.BlockSpec((RB, 1), lambda i: (i, 0)),
        pl.BlockSpec((1, D), lambda i: (0, 0)),
    ]
    args = [P, u, dinv, b]
    if not first:
        in_specs.append(pl.BlockSpec((RB, D), lambda i: (i, 0)))
        args.append(hprev)
    in_specs.append(pl.BlockSpec((D, D), lambda i: (0, 0)))
    args.append(Wnext)
    return pl.pallas_call(
        body,
        grid=(N // RB,),
        in_specs=in_specs,
        out_specs=[
            pl.BlockSpec((RB, D), lambda i: (i, 0)),
            pl.BlockSpec((RB, D), lambda i: (i, 0)),
        ],
        out_shape=[
            jax.ShapeDtypeStruct((N, D), _f32),
            jax.ShapeDtypeStruct((N, D), _f32),
        ],
    )(*args)


def _tc_last(P, u, dinv, b, hprev, Wf0pair):
    def body(P_ref, u_ref, dinv_ref, b_ref, hp_ref, w_ref, A_ref, B_ref):
        t = (P_ref[0] + P_ref[1] - u_ref[...]) * dinv_ref[...] + b_ref[...]
        h = jnp.maximum(t + hp_ref[...], 0.0)
        A_ref[...] = jnp.dot(h, w_ref[0], preferred_element_type=_f32)
        B_ref[...] = jnp.dot(h, w_ref[1], preferred_element_type=_f32)

    return pl.pallas_call(
        body,
        grid=(N // RB,),
        in_specs=[
            pl.BlockSpec((NC, RB, D), lambda i: (0, i, 0)),
            pl.BlockSpec((RB, D), lambda i: (i, 0)),
            pl.BlockSpec((RB, 1), lambda i: (i, 0)),
            pl.BlockSpec((1, D), lambda i: (0, 0)),
            pl.BlockSpec((RB, D), lambda i: (i, 0)),
            pl.BlockSpec((2, D, D), lambda i: (0, 0, 0)),
        ],
        out_specs=[
            pl.BlockSpec((RB, D), lambda i: (i, 0)),
            pl.BlockSpec((RB, D), lambda i: (i, 0)),
        ],
        out_shape=[
            jax.ShapeDtypeStruct((N, D), _f32),
            jax.ShapeDtypeStruct((N, D), _f32),
        ],
    )(P, u, dinv, b, hprev, Wf0pair)


def _tc_edge(gA, gB, b0, W1, b1, W2, b2):
    def body(gA_ref, gB_ref, b0_ref, w1_ref, b1_ref, w2_ref, b2_ref, out_ref):
        e = jnp.maximum(gA_ref[...] + gB_ref[...] + b0_ref[...], 0.0)
        e2 = jnp.maximum(
            jnp.dot(e, w1_ref[...], preferred_element_type=_f32) + b1_ref[...], 0.0
        )
        out_ref[...] = (
            jnp.dot(e2, w2_ref[...], preferred_element_type=_f32) + b2_ref[...]
        )

    return pl.pallas_call(
        body,
        grid=(E // RE,),
        in_specs=[
            pl.BlockSpec((RE, D), lambda i: (i, 0)),
            pl.BlockSpec((RE, D), lambda i: (i, 0)),
            pl.BlockSpec((1, D), lambda i: (0, 0)),
            pl.BlockSpec((D, D), lambda i: (0, 0)),
            pl.BlockSpec((1, D), lambda i: (0, 0)),
            pl.BlockSpec((D, 1), lambda i: (0, 0)),
            pl.BlockSpec((1, 1), lambda i: (0, 0)),
        ],
        out_specs=pl.BlockSpec((RE, 1), lambda i: (i, 0)),
        out_shape=jax.ShapeDtypeStruct((E, 1), _f32),
    )(gA, gB, b0, W1, b1, W2, b2)


def kernel(x, edge_index, Wc, bc, Wf0, bf0, Wf1, bf1, Wf2, bf2):
    src = edge_index[0].astype(jnp.int32)
    dst = edge_index[1].astype(jnp.int32)
    src3 = src.reshape(NW, NCH, CH)
    dst3 = dst.reshape(NW, NCH, CH)
    src2 = src.reshape(NW, EPW)
    dst2 = dst.reshape(NW, EPW)

    dinv = _tc_deg(_sc_degree(dst2))
    u = _tc_first(dinv, x, Wc[0])

    h = None
    for i in range(NCONV):
        P = _sc_segsum(u, src2, dst2)
        bi = bc[i].reshape(1, D)
        if i < NCONV - 1:
            h, u = _tc_mid(P, u, dinv, bi, h, Wc[i + 1], first=(i == 0))
        else:
            A, B = _tc_last(P, u, dinv, bi, h, Wf0.reshape(2, D, D))

    gA, gB = _sc_edge_gather(A, B, src3, dst3)
    return _tc_edge(
        gA, gB, bf0.reshape(1, D), Wf1, bf1.reshape(1, D), Wf2, bf2.reshape(1, 1)
    )


# speedup vs baseline: 16.7945x; 1.3223x over previous
import functools

import jax
import jax.numpy as jnp
from jax import lax
from jax.experimental import pallas as pl
from jax.experimental.pallas import tpu as pltpu
from jax.experimental.pallas import tpu_sc as plsc

N = 10000
D = 128
E = 320000
NCONV = 8

NC = 2
NS = 16
NW = NC * NS
EPW = E // NW
CH = 80
NCH = EPW // CH
NPT = 624
TOFF = NS * NPT
TAIL = N - TOFF

_f32 = jnp.float32


def _mesh():
    return plsc.VectorSubcoreMesh(core_axis_name="c", subcore_axis_name="s")


def _sc_degree(dst3):
    @functools.partial(
        pl.kernel,
        out_type=jax.ShapeDtypeStruct((NC, N), _f32),
        mesh=_mesh(),
        scratch_types=[
            pltpu.VMEM((NCH, CH), jnp.int32),
            pltpu.VMEM((CH,), _f32),
            pltpu.VMEM((N,), _f32),
            pltpu.VMEM_SHARED((N,), _f32),
        ],
    )
    def body(dst_hbm, out_hbm, dst_v, ones_v, zbuf, acc):
        c = lax.axis_index("c")
        s = lax.axis_index("s")
        wid = s * NC + c
        for j in range(CH // 16):
            ones_v[pl.ds(j * 16, 16)] = jnp.ones((16,), _f32)

        @pl.when(s == 0)
        def _zero():
            z16 = jnp.zeros((16,), _f32)

            def zstep(i, _):
                zbuf[pl.ds(i * 16, 16)] = z16
                return ()

            lax.fori_loop(0, N // 16, zstep, ())
            pltpu.sync_copy(zbuf, acc)

        pltpu.sync_copy(dst_hbm.at[wid], dst_v)
        plsc.subcore_barrier()

        def step(i, _):
            pltpu.sync_copy(ones_v, acc.at[dst_v.at[i]], add=True)
            return ()

        lax.fori_loop(0, NCH, step, ())
        plsc.subcore_barrier()

        @pl.when(s == 0)
        def _out():
            pltpu.sync_copy(acc, out_hbm.at[c])

    return body(dst3)


RING = 3


def _sc_segsum(u, pk2):
    @functools.partial(
        pl.kernel,
        out_type=jax.ShapeDtypeStruct((NC, N, D), _f32),
        mesh=_mesh(),
        scratch_types=[
            pltpu.VMEM((EPW,), jnp.int32),
            pltpu.VMEM((RING, CH), jnp.int32),
            pltpu.VMEM((RING, CH), jnp.int32),
            pltpu.VMEM((RING, CH, D), _f32),
            pltpu.VMEM_SHARED((N, D), _f32),
            [pltpu.SemaphoreType.DMA] * RING,
            [pltpu.SemaphoreType.DMA] * RING,
        ],
    )
    def body(u_hbm, pk_hbm, out_hbm, pk_v, sstage, dstage, rows_v,
             acc, gsems, ssems):
        c = lax.axis_index("c")
        s = lax.axis_index("s")
        wid = s * NC + c
        pltpu.sync_copy(u_hbm.at[pl.ds(s * NPT, NPT)], acc.at[pl.ds(s * NPT, NPT)])

        @pl.when(s == NS - 1)
        def _init_tail():
            pltpu.sync_copy(u_hbm.at[pl.ds(TOFF, TAIL)], acc.at[pl.ds(TOFF, TAIL)])

        pltpu.sync_copy(pk_hbm.at[wid], pk_v)
        plsc.subcore_barrier()

        def stage(r, cc):
            for t in range(CH // 16):
                pk = pk_v[pl.ds(cc * CH + t * 16, 16)]
                sstage[r, pl.ds(t * 16, 16)] = pk & 0xFFFF
                dstage[r, pl.ds(t * 16, 16)] = lax.shift_right_logical(pk, 16)

        def gather(r, g):
            pltpu.async_copy(u_hbm.at[sstage.at[r]], rows_v.at[r], g)

        def wait_gather(r, g):
            pltpu.make_async_copy(u_hbm.at[sstage.at[r]], rows_v.at[r], g).wait()

        def scatter(r, sm):
            pltpu.async_copy(rows_v.at[r], acc.at[dstage.at[r]], sm, add=True)

        def wait_scatter(r, sm):
            pltpu.make_async_copy(rows_v.at[r], acc.at[dstage.at[r]], sm).wait()

        def proc(cc, j, jn, gs, ss, issue, swait):
            if issue:
                if swait:
                    wait_scatter(jn, ss[jn])
                stage(jn, cc + 2)
                gather(jn, gs[jn])
            wait_gather(j, gs[j])
            scatter(j, ss[j])

        stage(0, 0)
        gather(0, gsems[0])
        stage(1, 1)
        gather(1, gsems[1])
        proc(0, 0, 2, gsems, ssems, issue=True, swait=False)

        def group(k, _):
            base = 3 * k
            proc(base + 1, 1, 0, gsems, ssems, issue=True, swait=True)
            proc(base + 2, 2, 1, gsems, ssems, issue=True, swait=True)
            proc(base + 3, 0, 2, gsems, ssems, issue=True, swait=True)
            return ()

        lax.fori_loop(0, (NCH - 5) // 3, group, ())
        proc(NCH - 4, 1, 0, gsems, ssems, issue=True, swait=True)
        proc(NCH - 3, 2, 1, gsems, ssems, issue=True, swait=True)
        proc(NCH - 2, 0, 2, gsems, ssems, issue=False, swait=False)
        proc(NCH - 1, 1, 0, gsems, ssems, issue=False, swait=False)
        wait_scatter(2, ssems[2])
        wait_scatter(0, ssems[0])
        wait_scatter(1, ssems[1])

        plsc.subcore_barrier()
        pltpu.sync_copy(acc.at[pl.ds(s * NPT, NPT)], out_hbm.at[c, pl.ds(s * NPT, NPT)])

        @pl.when(s == NS - 1)
        def _out_tail():
            pltpu.sync_copy(acc.at[pl.ds(TOFF, TAIL)], out_hbm.at[c, pl.ds(TOFF, TAIL)])

    return body(u, pk2)


def _sc_edge_gather(A, B, src3, dst3):
    @functools.partial(
        pl.kernel,
        out_type=(
            jax.ShapeDtypeStruct((E, D), _f32),
            jax.ShapeDtypeStruct((E, D), _f32),
        ),
        mesh=_mesh(),
        scratch_types=[
            pltpu.VMEM((NCH, CH), jnp.int32),
            pltpu.VMEM((NCH, CH), jnp.int32),
            pltpu.VMEM((2, CH, D), _f32),
            pltpu.VMEM((2, CH, D), _f32),
            pltpu.SemaphoreType.DMA,
            pltpu.SemaphoreType.DMA,
            pltpu.SemaphoreType.DMA,
            pltpu.SemaphoreType.DMA,
        ],
    )
    def body(A_hbm, B_hbm, row_hbm, col_hbm, gA_hbm, gB_hbm,
             row_v, col_v, bufA, bufB, sA0, sA1, sB0, sB1):
        sA = (sA0, sA1)
        sB = (sB0, sB1)
        c = lax.axis_index("c")
        s = lax.axis_index("s")
        wid = s * NC + c
        base = wid * EPW
        pltpu.sync_copy(row_hbm.at[wid], row_v)
        pltpu.sync_copy(col_hbm.at[wid], col_v)
        pltpu.async_copy(A_hbm.at[row_v.at[0]], bufA.at[0], sA[0])
        pltpu.async_copy(B_hbm.at[col_v.at[0]], bufB.at[0], sB[0])

        def group(k, _):
            for p in (1, 0):
                cc = 2 * k + (1 if p == 1 else 2)
                q = 1 - p
                pltpu.async_copy(A_hbm.at[row_v.at[cc]], bufA.at[p], sA[p])
                pltpu.async_copy(B_hbm.at[col_v.at[cc]], bufB.at[p], sB[p])
                pltpu.make_async_copy(
                    A_hbm.at[row_v.at[0]], bufA.at[q], sA[q]).wait()
                pltpu.make_async_copy(
                    B_hbm.at[col_v.at[0]], bufB.at[q], sB[q]).wait()
                eb = base + (cc - 1) * CH
                pltpu.sync_copy(bufA.at[q], gA_hbm.at[pl.ds(eb, CH)])
                pltpu.sync_copy(bufB.at[q], gB_hbm.at[pl.ds(eb, CH)])
            return ()

        lax.fori_loop(0, (NCH - 1) // 2, group, ())
        pltpu.make_async_copy(A_hbm.at[row_v.at[0]], bufA.at[0], sA[0]).wait()
        pltpu.make_async_copy(B_hbm.at[col_v.at[0]], bufB.at[0], sB[0]).wait()
        eb = base + (NCH - 1) * CH
        pltpu.sync_copy(bufA.at[0], gA_hbm.at[pl.ds(eb, CH)])
        pltpu.sync_copy(bufB.at[0], gB_hbm.at[pl.ds(eb, CH)])

    return body(A, B, src3, dst3)


RB = 2000
RE = 2000


def _tc_deg(degP):
    def body(degP_ref, dinv_ref):
        deg = 1.0 + jnp.sum(degP_ref[...], axis=0)
        dinv_ref[...] = (1.0 / jnp.sqrt(deg))[:, None]

    return pl.pallas_call(
        body,
        grid=(1,),
        in_specs=[pl.BlockSpec((NC, N), lambda i: (0, 0))],
        out_specs=pl.BlockSpec((N, 1), lambda i: (0, 0)),
        out_shape=jax.ShapeDtypeStruct((N, 1), _f32),
    )(degP)


def _tc_first(dinv, x, W0):
    def body(dinv_ref, x_ref, w_ref, u_ref):
        u_ref[...] = (
            jnp.dot(x_ref[...], w_ref[...], preferred_element_type=_f32)
            * dinv_ref[...]
        )

    return pl.pallas_call(
        body,
        grid=(N // RB,),
        in_specs=[
            pl.BlockSpec((RB, 1), lambda i: (i, 0)),
            pl.BlockSpec((RB, D), lambda i: (i, 0)),
            pl.BlockSpec((D, D), lambda i: (0, 0)),
        ],
        out_specs=pl.BlockSpec((RB, D), lambda i: (i, 0)),
        out_shape=jax.ShapeDtypeStruct((N, D), _f32),
    )(dinv, x, W0)


def _tc_mid(P, u, dinv, b, hprev, Wnext, first):
    def body(*refs):
        if first:
            P_ref, u_ref, dinv_ref, b_ref, w_ref, h_ref, un_ref = refs
        else:
            P_ref, u_ref, dinv_ref, b_ref, hp_ref, w_ref, h_ref, un_ref = refs
        t = (P_ref[0] + P_ref[1] - u_ref[...]) * dinv_ref[...] + b_ref[...]
        if not first:
            t = t + hp_ref[...]
        h = jnp.maximum(t, 0.0)
        h_ref[...] = h
        un_ref[...] = (
            jnp.dot(h, w_ref[...], preferred_element_type=_f32) * dinv_ref[...]
        )

    in_specs = [
        pl.BlockSpec((NC, RB, D), lambda i: (0, i, 0)),
        pl.BlockSpec((RB, D), lambda i: (i, 0)),
        pl.BlockSpec((RB, 1), lambda i: (i, 0)),
        pl.BlockSpec((1, D), lambda i: (0, 0)),
    ]
    args = [P, u, dinv, b]
    if not first:
        in_specs.append(pl.BlockSpec((RB, D), lambda i: (i, 0)))
        args.append(hprev)
    in_specs.append(pl.BlockSpec((D, D), lambda i: (0, 0)))
    args.append(Wnext)
    return pl.pallas_call(
        body,
        grid=(N // RB,),
        in_specs=in_specs,
        out_specs=[
            pl.BlockSpec((RB, D), lambda i: (i, 0)),
            pl.BlockSpec((RB, D), lambda i: (i, 0)),
        ],
        out_shape=[
            jax.ShapeDtypeStruct((N, D), _f32),
            jax.ShapeDtypeStruct((N, D), _f32),
        ],
    )(*args)


def _tc_last(P, u, dinv, b, hprev, Wf0pair):
    def body(P_ref, u_ref, dinv_ref, b_ref, hp_ref, w_ref, A_ref, B_ref):
        t = (P_ref[0] + P_ref[1] - u_ref[...]) * dinv_ref[...] + b_ref[...]
        h = jnp.maximum(t + hp_ref[...], 0.0)
        A_ref[...] = jnp.dot(h, w_ref[0], preferred_element_type=_f32)
        B_ref[...] = jnp.dot(h, w_ref[1], preferred_element_type=_f32)

    return pl.pallas_call(
        body,
        grid=(N // RB,),
        in_specs=[
            pl.BlockSpec((NC, RB, D), lambda i: (0, i, 0)),
            pl.BlockSpec((RB, D), lambda i: (i, 0)),
            pl.BlockSpec((RB, 1), lambda i: (i, 0)),
            pl.BlockSpec((1, D), lambda i: (0, 0)),
            pl.BlockSpec((RB, D), lambda i: (i, 0)),
            pl.BlockSpec((2, D, D), lambda i: (0, 0, 0)),
        ],
        out_specs=[
            pl.BlockSpec((RB, D), lambda i: (i, 0)),
            pl.BlockSpec((RB, D), lambda i: (i, 0)),
        ],
        out_shape=[
            jax.ShapeDtypeStruct((N, D), _f32),
            jax.ShapeDtypeStruct((N, D), _f32),
        ],
    )(P, u, dinv, b, hprev, Wf0pair)


def _tc_edge(gA, gB, b0, W1, b1, W2, b2):
    def body(gA_ref, gB_ref, b0_ref, w1_ref, b1_ref, w2_ref, b2_ref, out_ref):
        e = jnp.maximum(gA_ref[...] + gB_ref[...] + b0_ref[...], 0.0)
        e2 = jnp.maximum(
            jnp.dot(e, w1_ref[...], preferred_element_type=_f32) + b1_ref[...], 0.0
        )
        out_ref[...] = (
            jnp.dot(e2, w2_ref[...], preferred_element_type=_f32) + b2_ref[...]
        )

    return pl.pallas_call(
        body,
        grid=(E // RE,),
        in_specs=[
            pl.BlockSpec((RE, D), lambda i: (i, 0)),
            pl.BlockSpec((RE, D), lambda i: (i, 0)),
            pl.BlockSpec((1, D), lambda i: (0, 0)),
            pl.BlockSpec((D, D), lambda i: (0, 0)),
            pl.BlockSpec((1, D), lambda i: (0, 0)),
            pl.BlockSpec((D, 1), lambda i: (0, 0)),
            pl.BlockSpec((1, 1), lambda i: (0, 0)),
        ],
        out_specs=pl.BlockSpec((RE, 1), lambda i: (i, 0)),
        out_shape=jax.ShapeDtypeStruct((E, 1), _f32),
    )(gA, gB, b0, W1, b1, W2, b2)


def kernel(x, edge_index, Wc, bc, Wf0, bf0, Wf1, bf1, Wf2, bf2):
    src = edge_index[0].astype(jnp.int32)
    dst = edge_index[1].astype(jnp.int32)
    src3 = src.reshape(NW, NCH, CH)
    dst3 = dst.reshape(NW, NCH, CH)
    dst2 = dst.reshape(NW, EPW)
    pk2 = (src | (dst << 16)).reshape(NW, EPW)

    dinv = _tc_deg(_sc_degree(dst3))
    u = _tc_first(dinv, x, Wc[0])

    h = None
    for i in range(NCONV):
        P = _sc_segsum(u, pk2)
        bi = bc[i].reshape(1, D)
        if i < NCONV - 1:
            h, u = _tc_mid(P, u, dinv, bi, h, Wc[i + 1], first=(i == 0))
        else:
            A, B = _tc_last(P, u, dinv, bi, h, Wf0.reshape(2, D, D))

    gA, gB = _sc_edge_gather(A, B, src3, dst3)
    return _tc_edge(
        gA, gB, bf0.reshape(1, D), Wf1, bf1.reshape(1, D), Wf2, bf2.reshape(1, 1)
    )


# speedup vs baseline: 16.8057x; 1.0007x over previous
import functools

import jax
import jax.numpy as jnp
from jax import lax
from jax.experimental import pallas as pl
from jax.experimental.pallas import tpu as pltpu
from jax.experimental.pallas import tpu_sc as plsc

N = 10000
D = 128
E = 320000
NCONV = 8

NC = 2
NS = 16
NW = NC * NS
EPW = E // NW
CH = 80
NCH = EPW // CH
NPT = 624
TOFF = NS * NPT
TAIL = N - TOFF

_f32 = jnp.float32


def _mesh():
    return plsc.VectorSubcoreMesh(core_axis_name="c", subcore_axis_name="s")


def _sc_degree(dst3):
    @functools.partial(
        pl.kernel,
        out_type=jax.ShapeDtypeStruct((NC, N), _f32),
        mesh=_mesh(),
        scratch_types=[
            pltpu.VMEM((NCH, CH), jnp.int32),
            pltpu.VMEM((CH,), _f32),
            pltpu.VMEM((N,), _f32),
            pltpu.VMEM_SHARED((N,), _f32),
        ],
    )
    def body(dst_hbm, out_hbm, dst_v, ones_v, zbuf, acc):
        c = lax.axis_index("c")
        s = lax.axis_index("s")
        wid = s * NC + c
        for j in range(CH // 16):
            ones_v[pl.ds(j * 16, 16)] = jnp.ones((16,), _f32)

        @pl.when(s == 0)
        def _zero():
            z16 = jnp.zeros((16,), _f32)

            def zstep(i, _):
                zbuf[pl.ds(i * 16, 16)] = z16
                return ()

            lax.fori_loop(0, N // 16, zstep, ())
            pltpu.sync_copy(zbuf, acc)

        pltpu.sync_copy(dst_hbm.at[wid], dst_v)
        plsc.subcore_barrier()

        def step(i, _):
            pltpu.sync_copy(ones_v, acc.at[dst_v.at[i]], add=True)
            return ()

        lax.fori_loop(0, NCH, step, ())
        plsc.subcore_barrier()

        @pl.when(s == 0)
        def _out():
            pltpu.sync_copy(acc, out_hbm.at[c])

    return body(dst3)


RING = 3


def _sc_segsum(u, pk2):
    @functools.partial(
        pl.kernel,
        out_type=jax.ShapeDtypeStruct((NC, N, D), _f32),
        mesh=_mesh(),
        scratch_types=[
            pltpu.VMEM((EPW,), jnp.int32),
            pltpu.VMEM((RING, CH), jnp.int32),
            pltpu.VMEM((RING, CH), jnp.int32),
            pltpu.VMEM((RING, CH, D), _f32),
            pltpu.VMEM_SHARED((N, D), _f32),
            [pltpu.SemaphoreType.DMA] * RING,
            [pltpu.SemaphoreType.DMA] * RING,
        ],
    )
    def body(u_hbm, pk_hbm, out_hbm, pk_v, sstage, dstage, rows_v,
             acc, gsems, ssems):
        c = lax.axis_index("c")
        s = lax.axis_index("s")
        wid = s * NC + c
        pltpu.sync_copy(u_hbm.at[pl.ds(s * NPT, NPT)], acc.at[pl.ds(s * NPT, NPT)])

        @pl.when(s == NS - 1)
        def _init_tail():
            pltpu.sync_copy(u_hbm.at[pl.ds(TOFF, TAIL)], acc.at[pl.ds(TOFF, TAIL)])

        pltpu.sync_copy(pk_hbm.at[wid], pk_v)
        plsc.subcore_barrier()

        def stage(r, cc):
            for t in range(CH // 16):
                pk = pk_v[pl.ds(cc * CH + t * 16, 16)]
                sstage[r, pl.ds(t * 16, 16)] = pk & 0xFFFF
                dstage[r, pl.ds(t * 16, 16)] = lax.shift_right_logical(pk, 16)

        def gather(r, g):
            pltpu.async_copy(u_hbm.at[sstage.at[r]], rows_v.at[r], g)

        def wait_gather(r, g):
            pltpu.make_async_copy(u_hbm.at[sstage.at[r]], rows_v.at[r], g).wait()

        def scatter(r, sm):
            pltpu.async_copy(rows_v.at[r], acc.at[dstage.at[r]], sm, add=True)

        def wait_scatter(r, sm):
            pltpu.make_async_copy(rows_v.at[r], acc.at[dstage.at[r]], sm).wait()

        def proc(cc, j, jn, gs, ss, issue, swait):
            if issue:
                if swait:
                    wait_scatter(jn, ss[jn])
                stage(jn, cc + 2)
                gather(jn, gs[jn])
            wait_gather(j, gs[j])
            scatter(j, ss[j])

        stage(0, 0)
        gather(0, gsems[0])
        stage(1, 1)
        gather(1, gsems[1])
        proc(0, 0, 2, gsems, ssems, issue=True, swait=False)

        def group(k, _):
            base = 3 * k
            proc(base + 1, 1, 0, gsems, ssems, issue=True, swait=True)
            proc(base + 2, 2, 1, gsems, ssems, issue=True, swait=True)
            proc(base + 3, 0, 2, gsems, ssems, issue=True, swait=True)
            return ()

        lax.fori_loop(0, (NCH - 5) // 3, group, ())
        proc(NCH - 4, 1, 0, gsems, ssems, issue=True, swait=True)
        proc(NCH - 3, 2, 1, gsems, ssems, issue=True, swait=True)
        proc(NCH - 2, 0, 2, gsems, ssems, issue=False, swait=False)
        proc(NCH - 1, 1, 0, gsems, ssems, issue=False, swait=False)
        wait_scatter(2, ssems[2])
        wait_scatter(0, ssems[0])
        wait_scatter(1, ssems[1])

        plsc.subcore_barrier()
        pltpu.sync_copy(acc.at[pl.ds(s * NPT, NPT)], out_hbm.at[c, pl.ds(s * NPT, NPT)])

        @pl.when(s == NS - 1)
        def _out_tail():
            pltpu.sync_copy(acc.at[pl.ds(TOFF, TAIL)], out_hbm.at[c, pl.ds(TOFF, TAIL)])

    return body(u, pk2)


def _sc_edge_gather(A, B, src3, dst3):
    @functools.partial(
        pl.kernel,
        out_type=(
            jax.ShapeDtypeStruct((E, D), _f32),
            jax.ShapeDtypeStruct((E, D), _f32),
        ),
        mesh=_mesh(),
        scratch_types=[
            pltpu.VMEM((NCH, CH), jnp.int32),
            pltpu.VMEM((NCH, CH), jnp.int32),
            pltpu.VMEM((RING, CH, D), _f32),
            pltpu.VMEM((RING, CH, D), _f32),
            [pltpu.SemaphoreType.DMA] * RING,
            [pltpu.SemaphoreType.DMA] * RING,
            [pltpu.SemaphoreType.DMA] * RING,
            [pltpu.SemaphoreType.DMA] * RING,
        ],
    )
    def body(A_hbm, B_hbm, row_hbm, col_hbm, gA_hbm, gB_hbm,
             row_v, col_v, bufA, bufB, gsA, gsB, wsA, wsB):
        c = lax.axis_index("c")
        s = lax.axis_index("s")
        wid = s * NC + c
        base = wid * EPW
        pltpu.sync_copy(row_hbm.at[wid], row_v)
        pltpu.sync_copy(col_hbm.at[wid], col_v)

        def gathers(r, cc):
            pltpu.async_copy(A_hbm.at[row_v.at[cc]], bufA.at[r], gsA[r])
            pltpu.async_copy(B_hbm.at[col_v.at[cc]], bufB.at[r], gsB[r])

        def wait_gathers(r):
            pltpu.make_async_copy(A_hbm.at[row_v.at[0]], bufA.at[r], gsA[r]).wait()
            pltpu.make_async_copy(B_hbm.at[col_v.at[0]], bufB.at[r], gsB[r]).wait()

        def writes(r, cc):
            eb = base + cc * CH
            pltpu.async_copy(bufA.at[r], gA_hbm.at[pl.ds(eb, CH)], wsA[r])
            pltpu.async_copy(bufB.at[r], gB_hbm.at[pl.ds(eb, CH)], wsB[r])

        def wait_writes(r):
            pltpu.make_async_copy(bufA.at[r], gA_hbm.at[pl.ds(base, CH)], wsA[r]).wait()
            pltpu.make_async_copy(bufB.at[r], gB_hbm.at[pl.ds(base, CH)], wsB[r]).wait()

        def proc(cc, j, jn, issue, wwait):
            if issue:
                if wwait:
                    wait_writes(jn)
                gathers(jn, cc + 2)
            wait_gathers(j)
            writes(j, cc)

        gathers(0, 0)
        gathers(1, 1)
        proc(0, 0, 2, issue=True, wwait=False)

        def group(k, _):
            b3 = 3 * k
            proc(b3 + 1, 1, 0, issue=True, wwait=True)
            proc(b3 + 2, 2, 1, issue=True, wwait=True)
            proc(b3 + 3, 0, 2, issue=True, wwait=True)
            return ()

        lax.fori_loop(0, (NCH - 5) // 3, group, ())
        proc(NCH - 4, 1, 0, issue=True, wwait=True)
        proc(NCH - 3, 2, 1, issue=True, wwait=True)
        proc(NCH - 2, 0, 2, issue=False, wwait=False)
        proc(NCH - 1, 1, 0, issue=False, wwait=False)
        wait_writes(2)
        wait_writes(0)
        wait_writes(1)

    return body(A, B, src3, dst3)


RB = 2000
RE = 2000


def _tc_deg(degP):
    def body(degP_ref, dinv_ref):
        deg = 1.0 + jnp.sum(degP_ref[...], axis=0)
        dinv_ref[...] = (1.0 / jnp.sqrt(deg))[:, None]

    return pl.pallas_call(
        body,
        grid=(1,),
        in_specs=[pl.BlockSpec((NC, N), lambda i: (0, 0))],
        out_specs=pl.BlockSpec((N, 1), lambda i: (0, 0)),
        out_shape=jax.ShapeDtypeStruct((N, 1), _f32),
    )(degP)


def _tc_first(dinv, x, W0):
    def body(dinv_ref, x_ref, w_ref, u_ref):
        u_ref[...] = (
            jnp.dot(x_ref[...], w_ref[...], preferred_element_type=_f32)
            * dinv_ref[...]
        )

    return pl.pallas_call(
        body,
        grid=(N // RB,),
        in_specs=[
            pl.BlockSpec((RB, 1), lambda i: (i, 0)),
            pl.BlockSpec((RB, D), lambda i: (i, 0)),
            pl.BlockSpec((D, D), lambda i: (0, 0)),
        ],
        out_specs=pl.BlockSpec((RB, D), lambda i: (i, 0)),
        out_shape=jax.ShapeDtypeStruct((N, D), _f32),
    )(dinv, x, W0)


def _tc_mid(P, u, dinv, b, hprev, Wnext, first):
    def body(*refs):
        if first:
            P_ref, u_ref, dinv_ref, b_ref, w_ref, h_ref, un_ref = refs
        else:
            P_ref, u_ref, dinv_ref, b_ref, hp_ref, w_ref, h_ref, un_ref = refs
        t = (P_ref[0] + P_ref[1] - u_ref[...]) * dinv_ref[...] + b_ref[...]
        if not first:
            t = t + hp_ref[...]
        h = jnp.maximum(t, 0.0)
        h_ref[...] = h
        un_ref[...] = (
            jnp.dot(h, w_ref[...], preferred_element_type=_f32) * dinv_ref[...]
        )

    in_specs = [
        pl.BlockSpec((NC, RB, D), lambda i: (0, i, 0)),
        pl.BlockSpec((RB, D), lambda i: (i, 0)),
        pl.BlockSpec((RB, 1), lambda i: (i, 0)),
        pl.BlockSpec((1, D), lambda i: (0, 0)),
    ]
    args = [P, u, dinv, b]
    if not first:
        in_specs.append(pl.BlockSpec((RB, D), lambda i: (i, 0)))
        args.append(hprev)
    in_specs.append(pl.BlockSpec((D, D), lambda i: (0, 0)))
    args.append(Wnext)
    return pl.pallas_call(
        body,
        grid=(N // RB,),
        in_specs=in_specs,
        out_specs=[
            pl.BlockSpec((RB, D), lambda i: (i, 0)),
            pl.BlockSpec((RB, D), lambda i: (i, 0)),
        ],
        out_shape=[
            jax.ShapeDtypeStruct((N, D), _f32),
            jax.ShapeDtypeStruct((N, D), _f32),
        ],
    )(*args)


def _tc_last(P, u, dinv, b, hprev, Wf0pair):
    def body(P_ref, u_ref, dinv_ref, b_ref, hp_ref, w_ref, A_ref, B_ref):
        t = (P_ref[0] + P_ref[1] - u_ref[...]) * dinv_ref[...] + b_ref[...]
        h = jnp.maximum(t + hp_ref[...], 0.0)
        A_ref[...] = jnp.dot(h, w_ref[0], preferred_element_type=_f32)
        B_ref[...] = jnp.dot(h, w_ref[1], preferred_element_type=_f32)

    return pl.pallas_call(
        body,
        grid=(N // RB,),
        in_specs=[
            pl.BlockSpec((NC, RB, D), lambda i: (0, i, 0)),
            pl.BlockSpec((RB, D), lambda i: (i, 0)),
            pl.BlockSpec((RB, 1), lambda i: (i, 0)),
            pl.BlockSpec((1, D), lambda i: (0, 0)),
            pl.BlockSpec((RB, D), lambda i: (i, 0)),
            pl.BlockSpec((2, D, D), lambda i: (0, 0, 0)),
        ],
        out_specs=[
            pl.BlockSpec((RB, D), lambda i: (i, 0)),
            pl.BlockSpec((RB, D), lambda i: (i, 0)),
        ],
        out_shape=[
            jax.ShapeDtypeStruct((N, D), _f32),
            jax.ShapeDtypeStruct((N, D), _f32),
        ],
    )(P, u, dinv, b, hprev, Wf0pair)


def _tc_edge(gA, gB, b0, W1, b1, W2, b2):
    def body(gA_ref, gB_ref, b0_ref, w1_ref, b1_ref, w2_ref, b2_ref, out_ref):
        e = jnp.maximum(gA_ref[...] + gB_ref[...] + b0_ref[...], 0.0)
        e2 = jnp.maximum(
            jnp.dot(e, w1_ref[...], preferred_element_type=_f32) + b1_ref[...], 0.0
        )
        out_ref[...] = (
            jnp.dot(e2, w2_ref[...], preferred_element_type=_f32) + b2_ref[...]
        )

    return pl.pallas_call(
        body,
        grid=(E // RE,),
        in_specs=[
            pl.BlockSpec((RE, D), lambda i: (i, 0)),
            pl.BlockSpec((RE, D), lambda i: (i, 0)),
            pl.BlockSpec((1, D), lambda i: (0, 0)),
            pl.BlockSpec((D, D), lambda i: (0, 0)),
            pl.BlockSpec((1, D), lambda i: (0, 0)),
            pl.BlockSpec((D, 1), lambda i: (0, 0)),
            pl.BlockSpec((1, 1), lambda i: (0, 0)),
        ],
        out_specs=pl.BlockSpec((RE, 1), lambda i: (i, 0)),
        out_shape=jax.ShapeDtypeStruct((E, 1), _f32),
    )(gA, gB, b0, W1, b1, W2, b2)


def kernel(x, edge_index, Wc, bc, Wf0, bf0, Wf1, bf1, Wf2, bf2):
    src = edge_index[0].astype(jnp.int32)
    dst = edge_index[1].astype(jnp.int32)
    src3 = src.reshape(NW, NCH, CH)
    dst3 = dst.reshape(NW, NCH, CH)
    dst2 = dst.reshape(NW, EPW)
    pk2 = (src | (dst << 16)).reshape(NW, EPW)

    dinv = _tc_deg(_sc_degree(dst3))
    u = _tc_first(dinv, x, Wc[0])

    h = None
    for i in range(NCONV):
        P = _sc_segsum(u, pk2)
        bi = bc[i].reshape(1, D)
        if i < NCONV - 1:
            h, u = _tc_mid(P, u, dinv, bi, h, Wc[i + 1], first=(i == 0))
        else:
            A, B = _tc_last(P, u, dinv, bi, h, Wf0.reshape(2, D, D))

    gA, gB = _sc_edge_gather(A, B, src3, dst3)
    return _tc_edge(
        gA, gB, bf0.reshape(1, D), Wf1, bf1.reshape(1, D), Wf2, bf2.reshape(1, 1)
    )


# speedup vs baseline: 17.8130x; 1.0599x over previous
import functools

import jax
import jax.numpy as jnp
from jax import lax
from jax.experimental import pallas as pl
from jax.experimental.pallas import tpu as pltpu
from jax.experimental.pallas import tpu_sc as plsc

N = 10000
D = 128
E = 320000
NCONV = 8

NC = 2
NS = 16
NW = NC * NS
EPW = E // NW
CH = 80
NCH = EPW // CH
NPT = 624
TOFF = NS * NPT
TAIL = N - TOFF

_f32 = jnp.float32


def _mesh():
    return plsc.VectorSubcoreMesh(core_axis_name="c", subcore_axis_name="s")


def _sc_degree(dst3):
    @functools.partial(
        pl.kernel,
        out_type=jax.ShapeDtypeStruct((NC, N), _f32),
        mesh=_mesh(),
        scratch_types=[
            pltpu.VMEM((NCH, CH), jnp.int32),
            pltpu.VMEM((CH,), _f32),
            pltpu.VMEM((N,), _f32),
            pltpu.VMEM_SHARED((N,), _f32),
        ],
    )
    def body(dst_hbm, out_hbm, dst_v, ones_v, zbuf, acc):
        c = lax.axis_index("c")
        s = lax.axis_index("s")
        wid = s * NC + c
        for j in range(CH // 16):
            ones_v[pl.ds(j * 16, 16)] = jnp.ones((16,), _f32)

        @pl.when(s == 0)
        def _zero():
            z16 = jnp.zeros((16,), _f32)

            def zstep(i, _):
                zbuf[pl.ds(i * 16, 16)] = z16
                return ()

            lax.fori_loop(0, N // 16, zstep, ())
            pltpu.sync_copy(zbuf, acc)

        pltpu.sync_copy(dst_hbm.at[wid], dst_v)
        plsc.subcore_barrier()

        def step(i, _):
            pltpu.sync_copy(ones_v, acc.at[dst_v.at[i]], add=True)
            return ()

        lax.fori_loop(0, NCH, step, ())
        plsc.subcore_barrier()

        @pl.when(s == 0)
        def _out():
            pltpu.sync_copy(acc, out_hbm.at[c])

    return body(dst3)


RING = 3


def _sc_segsum(u, pk2):
    @functools.partial(
        pl.kernel,
        out_type=jax.ShapeDtypeStruct((NC, N, D), _f32),
        mesh=_mesh(),
        scratch_types=[
            pltpu.VMEM((EPW,), jnp.int32),
            pltpu.VMEM((RING, CH), jnp.int32),
            pltpu.VMEM((RING, CH), jnp.int32),
            pltpu.VMEM((RING, CH, D), _f32),
            pltpu.VMEM_SHARED((N, D), _f32),
            [pltpu.SemaphoreType.DMA] * RING,
            [pltpu.SemaphoreType.DMA] * RING,
        ],
    )
    def body(u_hbm, pk_hbm, out_hbm, pk_v, sstage, dstage, rows_v,
             acc, gsems, ssems):
        c = lax.axis_index("c")
        s = lax.axis_index("s")
        wid = s * NC + c
        pltpu.sync_copy(u_hbm.at[pl.ds(s * NPT, NPT)], acc.at[pl.ds(s * NPT, NPT)])

        @pl.when(s == NS - 1)
        def _init_tail():
            pltpu.sync_copy(u_hbm.at[pl.ds(TOFF, TAIL)], acc.at[pl.ds(TOFF, TAIL)])

        pltpu.sync_copy(pk_hbm.at[wid], pk_v)
        plsc.subcore_barrier()

        def stage(r, cc):
            for t in range(CH // 16):
                pk = pk_v[pl.ds(cc * CH + t * 16, 16)]
                sstage[r, pl.ds(t * 16, 16)] = pk & 0xFFFF
                dstage[r, pl.ds(t * 16, 16)] = lax.shift_right_logical(pk, 16)

        def gather(r, g):
            pltpu.async_copy(u_hbm.at[sstage.at[r]], rows_v.at[r], g)

        def wait_gather(r, g):
            pltpu.make_async_copy(u_hbm.at[sstage.at[r]], rows_v.at[r], g).wait()

        def scatter(r, sm):
            pltpu.async_copy(rows_v.at[r], acc.at[dstage.at[r]], sm, add=True)

        def wait_scatter(r, sm):
            pltpu.make_async_copy(rows_v.at[r], acc.at[dstage.at[r]], sm).wait()

        def proc(cc, j, jn, gs, ss, issue, swait):
            if issue:
                if swait:
                    wait_scatter(jn, ss[jn])
                stage(jn, cc + 2)
                gather(jn, gs[jn])
            wait_gather(j, gs[j])
            scatter(j, ss[j])

        stage(0, 0)
        gather(0, gsems[0])
        stage(1, 1)
        gather(1, gsems[1])
        proc(0, 0, 2, gsems, ssems, issue=True, swait=False)

        def group(k, _):
            base = 3 * k
            proc(base + 1, 1, 0, gsems, ssems, issue=True, swait=True)
            proc(base + 2, 2, 1, gsems, ssems, issue=True, swait=True)
            proc(base + 3, 0, 2, gsems, ssems, issue=True, swait=True)
            return ()

        lax.fori_loop(0, (NCH - 5) // 3, group, ())
        proc(NCH - 4, 1, 0, gsems, ssems, issue=True, swait=True)
        proc(NCH - 3, 2, 1, gsems, ssems, issue=True, swait=True)
        proc(NCH - 2, 0, 2, gsems, ssems, issue=False, swait=False)
        proc(NCH - 1, 1, 0, gsems, ssems, issue=False, swait=False)
        wait_scatter(2, ssems[2])
        wait_scatter(0, ssems[0])
        wait_scatter(1, ssems[1])

        plsc.subcore_barrier()
        pltpu.sync_copy(acc.at[pl.ds(s * NPT, NPT)], out_hbm.at[c, pl.ds(s * NPT, NPT)])

        @pl.when(s == NS - 1)
        def _out_tail():
            pltpu.sync_copy(acc.at[pl.ds(TOFF, TAIL)], out_hbm.at[c, pl.ds(TOFF, TAIL)])

    return body(u, pk2)


def _sc_edge_gather(A, B, src3, dst3):
    @functools.partial(
        pl.kernel,
        out_type=jax.ShapeDtypeStruct((E, D), _f32),
        mesh=_mesh(),
        scratch_types=[
            pltpu.VMEM((NCH, CH), jnp.int32),
            pltpu.VMEM((NCH, CH), jnp.int32),
            pltpu.VMEM((RING, CH, D), _f32),
            pltpu.VMEM((RING, CH, D), _f32),
            [pltpu.SemaphoreType.DMA] * RING,
            [pltpu.SemaphoreType.DMA] * RING,
            [pltpu.SemaphoreType.DMA] * RING,
        ],
    )
    def body(A_hbm, B_hbm, row_hbm, col_hbm, out_hbm,
             row_v, col_v, bufA, bufB, gsA, gsB, wsA):
        c = lax.axis_index("c")
        s = lax.axis_index("s")
        wid = s * NC + c
        base = wid * EPW
        pltpu.sync_copy(row_hbm.at[wid], row_v)
        pltpu.sync_copy(col_hbm.at[wid], col_v)

        def gathers(r, cc):
            pltpu.async_copy(A_hbm.at[row_v.at[cc]], bufA.at[r], gsA[r])
            pltpu.async_copy(B_hbm.at[col_v.at[cc]], bufB.at[r], gsB[r])

        def wait_gathers(r):
            pltpu.make_async_copy(A_hbm.at[row_v.at[0]], bufA.at[r], gsA[r]).wait()
            pltpu.make_async_copy(B_hbm.at[col_v.at[0]], bufB.at[r], gsB[r]).wait()

        def add_rows(r):
            def arow(q, _):
                for t in range(D // 16):
                    sl = pl.ds(t * 16, 16)
                    bufA[r, q, sl] = bufA[r, q, sl] + bufB[r, q, sl]
                return ()

            lax.fori_loop(0, CH, arow, ())

        def writes(r, cc):
            eb = base + cc * CH
            pltpu.async_copy(bufA.at[r], out_hbm.at[pl.ds(eb, CH)], wsA[r])

        def wait_writes(r):
            pltpu.make_async_copy(bufA.at[r], out_hbm.at[pl.ds(base, CH)], wsA[r]).wait()

        def proc(cc, j, jn, issue, wwait):
            if issue:
                if wwait:
                    wait_writes(jn)
                gathers(jn, cc + 2)
            wait_gathers(j)
            add_rows(j)
            writes(j, cc)

        gathers(0, 0)
        gathers(1, 1)
        proc(0, 0, 2, issue=True, wwait=False)

        def group(k, _):
            b3 = 3 * k
            proc(b3 + 1, 1, 0, issue=True, wwait=True)
            proc(b3 + 2, 2, 1, issue=True, wwait=True)
            proc(b3 + 3, 0, 2, issue=True, wwait=True)
            return ()

        lax.fori_loop(0, (NCH - 5) // 3, group, ())
        proc(NCH - 4, 1, 0, issue=True, wwait=True)
        proc(NCH - 3, 2, 1, issue=True, wwait=True)
        proc(NCH - 2, 0, 2, issue=False, wwait=False)
        proc(NCH - 1, 1, 0, issue=False, wwait=False)
        wait_writes(2)
        wait_writes(0)
        wait_writes(1)

    return body(A, B, src3, dst3)


RB = 2000
RE = 2000


def _tc_deg(degP):
    def body(degP_ref, dinv_ref):
        deg = 1.0 + jnp.sum(degP_ref[...], axis=0)
        dinv_ref[...] = (1.0 / jnp.sqrt(deg))[:, None]

    return pl.pallas_call(
        body,
        grid=(1,),
        in_specs=[pl.BlockSpec((NC, N), lambda i: (0, 0))],
        out_specs=pl.BlockSpec((N, 1), lambda i: (0, 0)),
        out_shape=jax.ShapeDtypeStruct((N, 1), _f32),
    )(degP)


def _tc_first(dinv, x, W0):
    def body(dinv_ref, x_ref, w_ref, u_ref):
        u_ref[...] = (
            jnp.dot(x_ref[...], w_ref[...], preferred_element_type=_f32)
            * dinv_ref[...]
        )

    return pl.pallas_call(
        body,
        grid=(N // RB,),
        in_specs=[
            pl.BlockSpec((RB, 1), lambda i: (i, 0)),
            pl.BlockSpec((RB, D), lambda i: (i, 0)),
            pl.BlockSpec((D, D), lambda i: (0, 0)),
        ],
        out_specs=pl.BlockSpec((RB, D), lambda i: (i, 0)),
        out_shape=jax.ShapeDtypeStruct((N, D), _f32),
    )(dinv, x, W0)


def _tc_mid(P, u, dinv, b, hprev, Wnext, first):
    def body(*refs):
        if first:
            P_ref, u_ref, dinv_ref, b_ref, w_ref, h_ref, un_ref = refs
        else:
            P_ref, u_ref, dinv_ref, b_ref, hp_ref, w_ref, h_ref, un_ref = refs
        t = (P_ref[0] + P_ref[1] - u_ref[...]) * dinv_ref[...] + b_ref[...]
        if not first:
            t = t + hp_ref[...]
        h = jnp.maximum(t, 0.0)
        h_ref[...] = h
        un_ref[...] = (
            jnp.dot(h, w_ref[...], preferred_element_type=_f32) * dinv_ref[...]
        )

    in_specs = [
        pl.BlockSpec((NC, RB, D), lambda i: (0, i, 0)),
        pl.BlockSpec((RB, D), lambda i: (i, 0)),
        pl.BlockSpec((RB, 1), lambda i: (i, 0)),
        pl.BlockSpec((1, D), lambda i: (0, 0)),
    ]
    args = [P, u, dinv, b]
    if not first:
        in_specs.append(pl.BlockSpec((RB, D), lambda i: (i, 0)))
        args.append(hprev)
    in_specs.append(pl.BlockSpec((D, D), lambda i: (0, 0)))
    args.append(Wnext)
    return pl.pallas_call(
        body,
        grid=(N // RB,),
        in_specs=in_specs,
        out_specs=[
            pl.BlockSpec((RB, D), lambda i: (i, 0)),
            pl.BlockSpec((RB, D), lambda i: (i, 0)),
        ],
        out_shape=[
            jax.ShapeDtypeStruct((N, D), _f32),
            jax.ShapeDtypeStruct((N, D), _f32),
        ],
    )(*args)


def _tc_last(P, u, dinv, b, hprev, Wf0pair):
    def body(P_ref, u_ref, dinv_ref, b_ref, hp_ref, w_ref, A_ref, B_ref):
        t = (P_ref[0] + P_ref[1] - u_ref[...]) * dinv_ref[...] + b_ref[...]
        h = jnp.maximum(t + hp_ref[...], 0.0)
        A_ref[...] = jnp.dot(h, w_ref[0], preferred_element_type=_f32)
        B_ref[...] = jnp.dot(h, w_ref[1], preferred_element_type=_f32)

    return pl.pallas_call(
        body,
        grid=(N // RB,),
        in_specs=[
            pl.BlockSpec((NC, RB, D), lambda i: (0, i, 0)),
            pl.BlockSpec((RB, D), lambda i: (i, 0)),
            pl.BlockSpec((RB, 1), lambda i: (i, 0)),
            pl.BlockSpec((1, D), lambda i: (0, 0)),
            pl.BlockSpec((RB, D), lambda i: (i, 0)),
            pl.BlockSpec((2, D, D), lambda i: (0, 0, 0)),
        ],
        out_specs=[
            pl.BlockSpec((RB, D), lambda i: (i, 0)),
            pl.BlockSpec((RB, D), lambda i: (i, 0)),
        ],
        out_shape=[
            jax.ShapeDtypeStruct((N, D), _f32),
            jax.ShapeDtypeStruct((N, D), _f32),
        ],
    )(P, u, dinv, b, hprev, Wf0pair)


def _tc_edge(epre, b0, W1, b1, W2, b2):
    def body(ep_ref, b0_ref, w1_ref, b1_ref, w2_ref, b2_ref, out_ref):
        e = jnp.maximum(ep_ref[...] + b0_ref[...], 0.0)
        e2 = jnp.maximum(
            jnp.dot(e, w1_ref[...], preferred_element_type=_f32) + b1_ref[...], 0.0
        )
        out_ref[...] = (
            jnp.dot(e2, w2_ref[...], preferred_element_type=_f32) + b2_ref[...]
        )

    return pl.pallas_call(
        body,
        grid=(E // RE,),
        in_specs=[
            pl.BlockSpec((RE, D), lambda i: (i, 0)),
            pl.BlockSpec((1, D), lambda i: (0, 0)),
            pl.BlockSpec((D, D), lambda i: (0, 0)),
            pl.BlockSpec((1, D), lambda i: (0, 0)),
            pl.BlockSpec((D, 1), lambda i: (0, 0)),
            pl.BlockSpec((1, 1), lambda i: (0, 0)),
        ],
        out_specs=pl.BlockSpec((RE, 1), lambda i: (i, 0)),
        out_shape=jax.ShapeDtypeStruct((E, 1), _f32),
    )(epre, b0, W1, b1, W2, b2)


def kernel(x, edge_index, Wc, bc, Wf0, bf0, Wf1, bf1, Wf2, bf2):
    src = edge_index[0].astype(jnp.int32)
    dst = edge_index[1].astype(jnp.int32)
    src3 = src.reshape(NW, NCH, CH)
    dst3 = dst.reshape(NW, NCH, CH)
    dst2 = dst.reshape(NW, EPW)
    pk2 = (src | (dst << 16)).reshape(NW, EPW)

    dinv = _tc_deg(_sc_degree(dst3))
    u = _tc_first(dinv, x, Wc[0])

    h = None
    for i in range(NCONV):
        P = _sc_segsum(u, pk2)
        bi = bc[i].reshape(1, D)
        if i < NCONV - 1:
            h, u = _tc_mid(P, u, dinv, bi, h, Wc[i + 1], first=(i == 0))
        else:
            A, B = _tc_last(P, u, dinv, bi, h, Wf0.reshape(2, D, D))

    epre = _sc_edge_gather(A, B, src3, dst3)
    return _tc_edge(
        epre, bf0.reshape(1, D), Wf1, bf1.reshape(1, D), Wf2, bf2.reshape(1, 1)
    )


# speedup vs baseline: 17.8268x; 1.0008x over previous
import functools

import jax
import jax.numpy as jnp
from jax import lax
from jax.experimental import pallas as pl
from jax.experimental.pallas import tpu as pltpu
from jax.experimental.pallas import tpu_sc as plsc

N = 10000
D = 128
E = 320000
NCONV = 8

NC = 2
NS = 16
NW = NC * NS
EPW = E // NW
CH = 80
NCH = EPW // CH
NPT = 624
TOFF = NS * NPT
TAIL = N - TOFF

_f32 = jnp.float32


def _mesh():
    return plsc.VectorSubcoreMesh(core_axis_name="c", subcore_axis_name="s")


def _sc_degree(dst3):
    @functools.partial(
        pl.kernel,
        out_type=jax.ShapeDtypeStruct((NC, N), _f32),
        mesh=_mesh(),
        scratch_types=[
            pltpu.VMEM((NCH, CH), jnp.int32),
            pltpu.VMEM((CH,), _f32),
            pltpu.VMEM((N,), _f32),
            pltpu.VMEM_SHARED((N,), _f32),
        ],
    )
    def body(dst_hbm, out_hbm, dst_v, ones_v, zbuf, acc):
        c = lax.axis_index("c")
        s = lax.axis_index("s")
        wid = s * NC + c
        for j in range(CH // 16):
            ones_v[pl.ds(j * 16, 16)] = jnp.ones((16,), _f32)

        @pl.when(s == 0)
        def _zero():
            z16 = jnp.zeros((16,), _f32)

            def zstep(i, _):
                zbuf[pl.ds(i * 16, 16)] = z16
                return ()

            lax.fori_loop(0, N // 16, zstep, ())
            pltpu.sync_copy(zbuf, acc)

        pltpu.sync_copy(dst_hbm.at[wid], dst_v)
        plsc.subcore_barrier()

        def step(i, _):
            pltpu.sync_copy(ones_v, acc.at[dst_v.at[i]], add=True)
            return ()

        lax.fori_loop(0, NCH, step, ())
        plsc.subcore_barrier()

        @pl.when(s == 0)
        def _out():
            pltpu.sync_copy(acc, out_hbm.at[c])

    return body(dst3)


RING = 3


def _sc_segsum(u, pk2):
    @functools.partial(
        pl.kernel,
        out_type=jax.ShapeDtypeStruct((NC, N, D), _f32),
        mesh=_mesh(),
        scratch_types=[
            pltpu.VMEM((EPW,), jnp.int32),
            pltpu.VMEM((RING, CH), jnp.int32),
            pltpu.VMEM((RING, CH), jnp.int32),
            pltpu.VMEM((RING, CH, D), _f32),
            pltpu.VMEM_SHARED((N, D), _f32),
            [pltpu.SemaphoreType.DMA] * RING,
            [pltpu.SemaphoreType.DMA] * RING,
        ],
    )
    def body(u_hbm, pk_hbm, out_hbm, pk_v, sstage, dstage, rows_v,
             acc, gsems, ssems):
        c = lax.axis_index("c")
        s = lax.axis_index("s")
        wid = s * NC + c
        pltpu.sync_copy(u_hbm.at[pl.ds(s * NPT, NPT)], acc.at[pl.ds(s * NPT, NPT)])

        @pl.when(s == NS - 1)
        def _init_tail():
            pltpu.sync_copy(u_hbm.at[pl.ds(TOFF, TAIL)], acc.at[pl.ds(TOFF, TAIL)])

        pltpu.sync_copy(pk_hbm.at[wid], pk_v)
        plsc.subcore_barrier()

        def stage(r, cc):
            for t in range(CH // 16):
                pk = pk_v[pl.ds(cc * CH + t * 16, 16)]
                sstage[r, pl.ds(t * 16, 16)] = pk & 0xFFFF
                dstage[r, pl.ds(t * 16, 16)] = lax.shift_right_logical(pk, 16)

        def gather(r, g):
            pltpu.async_copy(u_hbm.at[sstage.at[r]], rows_v.at[r], g)

        def wait_gather(r, g):
            pltpu.make_async_copy(u_hbm.at[sstage.at[r]], rows_v.at[r], g).wait()

        def scatter(r, sm):
            pltpu.async_copy(rows_v.at[r], acc.at[dstage.at[r]], sm, add=True)

        def wait_scatter(r, sm):
            pltpu.make_async_copy(rows_v.at[r], acc.at[dstage.at[r]], sm).wait()

        def proc(cc, j, jn, gs, ss, issue, swait):
            if issue:
                if swait:
                    wait_scatter(jn, ss[jn])
                stage(jn, cc + 2)
                gather(jn, gs[jn])
            wait_gather(j, gs[j])
            scatter(j, ss[j])

        stage(0, 0)
        gather(0, gsems[0])
        stage(1, 1)
        gather(1, gsems[1])
        proc(0, 0, 2, gsems, ssems, issue=True, swait=False)

        def group(k, _):
            base = 3 * k
            proc(base + 1, 1, 0, gsems, ssems, issue=True, swait=True)
            proc(base + 2, 2, 1, gsems, ssems, issue=True, swait=True)
            proc(base + 3, 0, 2, gsems, ssems, issue=True, swait=True)
            return ()

        lax.fori_loop(0, (NCH - 5) // 3, group, ())
        proc(NCH - 4, 1, 0, gsems, ssems, issue=True, swait=True)
        proc(NCH - 3, 2, 1, gsems, ssems, issue=True, swait=True)
        proc(NCH - 2, 0, 2, gsems, ssems, issue=False, swait=False)
        proc(NCH - 1, 1, 0, gsems, ssems, issue=False, swait=False)
        wait_scatter(2, ssems[2])
        wait_scatter(0, ssems[0])
        wait_scatter(1, ssems[1])

        plsc.subcore_barrier()
        pltpu.sync_copy(acc.at[pl.ds(s * NPT, NPT)], out_hbm.at[c, pl.ds(s * NPT, NPT)])

        @pl.when(s == NS - 1)
        def _out_tail():
            pltpu.sync_copy(acc.at[pl.ds(TOFF, TAIL)], out_hbm.at[c, pl.ds(TOFF, TAIL)])

    return body(u, pk2)


def _sc_edge_gather(A, B, src3, dst3):
    @functools.partial(
        pl.kernel,
        out_type=jax.ShapeDtypeStruct((E, D), _f32),
        mesh=_mesh(),
        scratch_types=[
            pltpu.VMEM((NCH, CH), jnp.int32),
            pltpu.VMEM((NCH, CH), jnp.int32),
            pltpu.VMEM((RING, CH, D), _f32),
            pltpu.VMEM((RING, CH, D), _f32),
            [pltpu.SemaphoreType.DMA] * RING,
            [pltpu.SemaphoreType.DMA] * RING,
            [pltpu.SemaphoreType.DMA] * RING,
        ],
    )
    def body(A_hbm, B_hbm, row_hbm, col_hbm, out_hbm,
             row_v, col_v, bufA, bufB, gsA, gsB, wsA):
        c = lax.axis_index("c")
        s = lax.axis_index("s")
        wid = s * NC + c
        base = wid * EPW
        pltpu.sync_copy(row_hbm.at[wid], row_v)
        pltpu.sync_copy(col_hbm.at[wid], col_v)

        def gathers(r, cc):
            pltpu.async_copy(A_hbm.at[row_v.at[cc]], bufA.at[r], gsA[r])
            pltpu.async_copy(B_hbm.at[col_v.at[cc]], bufB.at[r], gsB[r])

        def wait_gathers(r):
            pltpu.make_async_copy(A_hbm.at[row_v.at[0]], bufA.at[r], gsA[r]).wait()
            pltpu.make_async_copy(B_hbm.at[col_v.at[0]], bufB.at[r], gsB[r]).wait()

        def add_rows(r):
            def arow(q, _):
                for t in range(D // 16):
                    sl = pl.ds(t * 16, 16)
                    bufA[r, q, sl] = bufA[r, q, sl] + bufB[r, q, sl]
                return ()

            lax.fori_loop(0, CH, arow, ())

        def writes(r, cc):
            eb = base + cc * CH
            pltpu.async_copy(bufA.at[r], out_hbm.at[pl.ds(eb, CH)], wsA[r])

        def wait_writes(r):
            pltpu.make_async_copy(bufA.at[r], out_hbm.at[pl.ds(base, CH)], wsA[r]).wait()

        def proc(cc, j, jn, issue, wwait):
            if issue:
                if wwait:
                    wait_writes(jn)
                gathers(jn, cc + 2)
            wait_gathers(j)
            add_rows(j)
            writes(j, cc)

        gathers(0, 0)
        gathers(1, 1)
        proc(0, 0, 2, issue=True, wwait=False)

        def group(k, _):
            b3 = 3 * k
            proc(b3 + 1, 1, 0, issue=True, wwait=True)
            proc(b3 + 2, 2, 1, issue=True, wwait=True)
            proc(b3 + 3, 0, 2, issue=True, wwait=True)
            return ()

        lax.fori_loop(0, (NCH - 5) // 3, group, ())
        proc(NCH - 4, 1, 0, issue=True, wwait=True)
        proc(NCH - 3, 2, 1, issue=True, wwait=True)
        proc(NCH - 2, 0, 2, issue=False, wwait=False)
        proc(NCH - 1, 1, 0, issue=False, wwait=False)
        wait_writes(2)
        wait_writes(0)
        wait_writes(1)

    return body(A, B, src3, dst3)


RB = 2000
RE = 2000


def _tc_deg(degP):
    def body(degP_ref, dinv_ref):
        deg = 1.0 + jnp.sum(degP_ref[...], axis=0)
        dinv_ref[...] = (1.0 / jnp.sqrt(deg))[:, None]

    return pl.pallas_call(
        body,
        grid=(1,),
        in_specs=[pl.BlockSpec((NC, N), lambda i: (0, 0))],
        out_specs=pl.BlockSpec((N, 1), lambda i: (0, 0)),
        out_shape=jax.ShapeDtypeStruct((N, 1), _f32),
    )(degP)


def _tc_first(dinv, x, W0):
    def body(dinv_ref, x_ref, w_ref, u_ref):
        u_ref[...] = (
            jnp.dot(x_ref[...], w_ref[...], preferred_element_type=_f32)
            * dinv_ref[...]
        )

    return pl.pallas_call(
        body,
        grid=(N // RB,),
        in_specs=[
            pl.BlockSpec((RB, 1), lambda i: (i, 0)),
            pl.BlockSpec((RB, D), lambda i: (i, 0)),
            pl.BlockSpec((D, D), lambda i: (0, 0)),
        ],
        out_specs=pl.BlockSpec((RB, D), lambda i: (i, 0)),
        out_shape=jax.ShapeDtypeStruct((N, D), _f32),
    )(dinv, x, W0)


def _tc_mid(P, u, dinv, b, hprev, Wnext, first):
    def body(*refs):
        if first:
            P_ref, u_ref, dinv_ref, b_ref, w_ref, h_ref, un_ref = refs
        else:
            P_ref, u_ref, dinv_ref, b_ref, hp_ref, w_ref, h_ref, un_ref = refs
        t = (P_ref[0] + P_ref[1] - u_ref[...]) * dinv_ref[...] + b_ref[...]
        if not first:
            t = t + hp_ref[...]
        h = jnp.maximum(t, 0.0)
        h_ref[...] = h
        un_ref[...] = (
            jnp.dot(h, w_ref[...], preferred_element_type=_f32) * dinv_ref[...]
        )

    in_specs = [
        pl.BlockSpec((NC, RB, D), lambda i: (0, i, 0)),
        pl.BlockSpec((RB, D), lambda i: (i, 0)),
        pl.BlockSpec((RB, 1), lambda i: (i, 0)),
        pl.BlockSpec((1, D), lambda i: (0, 0)),
    ]
    args = [P, u, dinv, b]
    if not first:
        in_specs.append(pl.BlockSpec((RB, D), lambda i: (i, 0)))
        args.append(hprev)
    in_specs.append(pl.BlockSpec((D, D), lambda i: (0, 0)))
    args.append(Wnext)
    return pl.pallas_call(
        body,
        grid=(N // RB,),
        in_specs=in_specs,
        out_specs=[
            pl.BlockSpec((RB, D), lambda i: (i, 0)),
            pl.BlockSpec((RB, D), lambda i: (i, 0)),
        ],
        out_shape=[
            jax.ShapeDtypeStruct((N, D), _f32),
            jax.ShapeDtypeStruct((N, D), _f32),
        ],
    )(*args)


def _tc_last(P, u, dinv, b, hprev, Wf0pair):
    def body(P_ref, u_ref, dinv_ref, b_ref, hp_ref, w_ref, A_ref, B_ref):
        t = (P_ref[0] + P_ref[1] - u_ref[...]) * dinv_ref[...] + b_ref[...]
        h = jnp.maximum(t + hp_ref[...], 0.0)
        A_ref[...] = jnp.dot(h, w_ref[0], preferred_element_type=_f32)
        B_ref[...] = jnp.dot(h, w_ref[1], preferred_element_type=_f32)

    return pl.pallas_call(
        body,
        grid=(N // RB,),
        in_specs=[
            pl.BlockSpec((NC, RB, D), lambda i: (0, i, 0)),
            pl.BlockSpec((RB, D), lambda i: (i, 0)),
            pl.BlockSpec((RB, 1), lambda i: (i, 0)),
            pl.BlockSpec((1, D), lambda i: (0, 0)),
            pl.BlockSpec((RB, D), lambda i: (i, 0)),
            pl.BlockSpec((2, D, D), lambda i: (0, 0, 0)),
        ],
        out_specs=[
            pl.BlockSpec((RB, D), lambda i: (i, 0)),
            pl.BlockSpec((RB, D), lambda i: (i, 0)),
        ],
        out_shape=[
            jax.ShapeDtypeStruct((N, D), _f32),
            jax.ShapeDtypeStruct((N, D), _f32),
        ],
    )(P, u, dinv, b, hprev, Wf0pair)


def _tc_edge(epre, b0, W1, b1, W2, b2):
    def body(ep_ref, b0_ref, w1_ref, b1_ref, w2_ref, b2_ref, out_ref):
        e = jnp.maximum(ep_ref[...] + b0_ref[...], 0.0)
        e2 = jnp.maximum(
            jnp.dot(e, w1_ref[...], preferred_element_type=_f32) + b1_ref[...], 0.0
        )
        out_ref[...] = (
            jnp.dot(e2, w2_ref[...], preferred_element_type=_f32) + b2_ref[...]
        )

    return pl.pallas_call(
        body,
        grid=(E // RE,),
        in_specs=[
            pl.BlockSpec((RE, D), lambda i: (i, 0)),
            pl.BlockSpec((1, D), lambda i: (0, 0)),
            pl.BlockSpec((D, D), lambda i: (0, 0)),
            pl.BlockSpec((1, D), lambda i: (0, 0)),
            pl.BlockSpec((D, 1), lambda i: (0, 0)),
            pl.BlockSpec((1, 1), lambda i: (0, 0)),
        ],
        out_specs=pl.BlockSpec((RE, 1), lambda i: (i, 0)),
        out_shape=jax.ShapeDtypeStruct((E, 1), _f32),
    )(epre, b0, W1, b1, W2, b2)


def kernel(x, edge_index, Wc, bc, Wf0, bf0, Wf1, bf1, Wf2, bf2):
    src = edge_index[0].astype(jnp.int32)
    dst = edge_index[1].astype(jnp.int32)
    src3 = src.reshape(NW, NCH, CH)
    dst3 = dst.reshape(NW, NCH, CH)
    pk2 = (src | (dst << 16)).reshape(NW, EPW)

    dinv = _tc_deg(_sc_degree(dst3))
    u = _tc_first(dinv, x, Wc[0])

    h = None
    for i in range(NCONV):
        P = _sc_segsum(u, pk2)
        bi = bc[i].reshape(1, D)
        if i < NCONV - 1:
            h, u = _tc_mid(P, u, dinv, bi, h, Wc[i + 1], first=(i == 0))
        else:
            A, B = _tc_last(P, u, dinv, bi, h, Wf0.reshape(2, D, D))

    epre = _sc_edge_gather(A, B, src3, dst3)
    return _tc_edge(
        epre, bf0.reshape(1, D), Wf1, bf1.reshape(1, D), Wf2, bf2.reshape(1, 1)
    )
